# baseline (speedup 1.0000x reference)
import jax
import jax.numpy as jnp
from jax.experimental import pallas as pl

N = 10000
HID = 128


def _layer_norm(h, eps=1e-5):
    mu = jnp.mean(h, axis=-1, keepdims=True)
    var = jnp.var(h, axis=-1, keepdims=True)
    return (h - mu) / jnp.sqrt(var + eps)


def _final_linear_kernel(h_ref, w_ref, b_ref, o_ref):
    o_ref[...] = h_ref[...] @ w_ref[...] + b_ref[...]


def kernel(x, edge_index, edge_attr, emb_W, emb_b, c1_W0, c1_W1, c1_b, c2_W0, c2_W1, c2_b, lin_W, lin_b):
    src = edge_index[0]
    dst = edge_index[1]
    z = jnp.concatenate([x[dst], x[src], edge_attr], axis=-1)
    m = jax.nn.relu(z @ emb_W + emb_b)
    h = jax.ops.segment_max(m, dst, num_segments=N)
    h = jnp.where(jnp.isneginf(h), 0.0, h)
    w = jnp.where(src != dst, 1.0, 0.0).astype(jnp.float32)
    deg = jax.ops.segment_sum(w, src, num_segments=N)
    dis = jnp.where(deg > 0, 1.0 / jnp.sqrt(jnp.where(deg > 0, deg, 1.0)), 0.0)
    norm = -dis[src] * w * dis[dst]

    def cheb(h, W0, W1, b):
        Tx1 = jax.ops.segment_sum(norm[:, None] * h[src], dst, num_segments=N)
        return h @ W0 + Tx1 @ W1 + b

    h = jax.nn.relu(_layer_norm(cheb(h, c1_W0, c1_W1, c1_b)))
    h = jax.nn.relu(_layer_norm(cheb(h, c2_W0, c2_W1, c2_b)))
    out = pl.pallas_call(
        _final_linear_kernel,
        out_shape=jax.ShapeDtypeStruct((N, 1), jnp.float32),
    )(h, lin_W, lin_b)
    return out[:, 0]


# speedup vs baseline: 5.3168x; 5.3168x over previous
import functools

import jax
import jax.numpy as jnp
from jax import lax
from jax.experimental import pallas as pl
from jax.experimental.pallas import tpu as pltpu
from jax.experimental.pallas import tpu_sc as plsc

N = 10000
E = 320000
D_FEAT = 128
D_EDGE = 16
EMB = 32
HID = 128

NC, NS, L = 2, 16, 16
NW = NC * NS
NP = 320
NPAD = NW * NP
GPAD = 10016
CE = 2000
LISTCAP = 12288
KB = 128
NBLK = E // KB

_SC_PARAMS = pltpu.CompilerParams(use_tc_tiling_on_sc=False,
                                  needs_layout_passes=False)


def _mesh():
    return plsc.VectorSubcoreMesh(core_axis_name="c", subcore_axis_name="s",
                                  num_cores=NC, num_subcores=NS)


def _node_proj_body(x_ref, w_ref, xa_ref, xb_ref):
    x = x_ref[...]
    w = w_ref[...]
    xa_ref[...] = jax.lax.dot_general(x, w[0:D_FEAT, :], (((1,), (0,)), ((), ())),
                                      preferred_element_type=jnp.float32)
    xb_ref[...] = jax.lax.dot_general(x, w[D_FEAT:2 * D_FEAT, :], (((1,), (0,)), ((), ())),
                                      preferred_element_type=jnp.float32)


def _node_proj(x, emb_W):
    blk = 1000
    return pl.pallas_call(
        _node_proj_body,
        grid=(N // blk,),
        in_specs=[pl.BlockSpec((blk, D_FEAT), lambda i: (i, 0)),
                  pl.BlockSpec((2 * D_FEAT + D_EDGE, EMB), lambda i: (0, 0))],
        out_specs=[pl.BlockSpec((blk, EMB), lambda i: (i, 0)),
                   pl.BlockSpec((blk, EMB), lambda i: (i, 0))],
        out_shape=[jax.ShapeDtypeStruct((N, EMB), jnp.float32),
                   jax.ShapeDtypeStruct((N, EMB), jnp.float32)],
    )(x, emb_W)


def _edge_proj_body(ea_ref, w_ref, b_ref, o_ref):
    w = w_ref[...]
    o_ref[...] = jax.lax.dot_general(
        ea_ref[...], w[2 * D_FEAT:, :], (((1,), (0,)), ((), ())),
        preferred_element_type=jnp.float32) + b_ref[...]


def _edge_proj(edge_attr, emb_W, emb_b):
    blk = 8000
    emb_b = emb_b.reshape(1, EMB)
    return pl.pallas_call(
        _edge_proj_body,
        grid=(E // blk,),
        in_specs=[pl.BlockSpec((blk, D_EDGE), lambda i: (i, 0)),
                  pl.BlockSpec((2 * D_FEAT + D_EDGE, EMB), lambda i: (0, 0)),
                  pl.BlockSpec((1, EMB), lambda i: (0, 0))],
        out_specs=pl.BlockSpec((blk, EMB), lambda i: (i, 0)),
        out_shape=jax.ShapeDtypeStruct((E, EMB), jnp.float32),
    )(edge_attr, emb_W, emb_b)


def _mid1_body(h_ref, deg_ref, dis_ref, g_ref):
    deg = deg_ref[...]
    dis = jnp.where(deg > 0, jax.lax.rsqrt(jnp.where(deg > 0, deg, 1.0)), 0.0)
    dis_ref[...] = dis
    g_ref[0:N, :] = dis * h_ref[...]
    g_ref[N:GPAD, :] = jnp.zeros((GPAD - N, EMB), jnp.float32)


def _mid1(h, deg):
    return pl.pallas_call(
        _mid1_body,
        out_shape=[jax.ShapeDtypeStruct((N, 1), jnp.float32),
                   jax.ShapeDtypeStruct((GPAD, EMB), jnp.float32)],
    )(h, deg)


def _layer_norm(p):
    mu = jnp.mean(p, axis=-1, keepdims=True)
    var = jnp.mean((p - mu) ** 2, axis=-1, keepdims=True)
    return (p - mu) / jnp.sqrt(var + 1e-5)


def _comb1_body(h_ref, p_ref, dis_ref, w0_ref, w1_ref, b_ref, h1_ref, g_ref):
    h = h_ref[...]
    dis = dis_ref[...]
    t1 = p_ref[0, 0:N, :] + p_ref[1, 0:N, :]
    pre = (jax.lax.dot_general(h, w0_ref[...], (((1,), (0,)), ((), ())),
                               preferred_element_type=jnp.float32)
           - dis * jax.lax.dot_general(t1, w1_ref[...], (((1,), (0,)), ((), ())),
                                       preferred_element_type=jnp.float32)
           + b_ref[...])
    h1 = jnp.maximum(_layer_norm(pre), 0.0)
    h1_ref[...] = h1
    g_ref[0:N, :] = dis * h1
    g_ref[N:GPAD, :] = jnp.zeros((GPAD - N, HID), jnp.float32)


def _comb1(h, p, dis, w0, w1, b):
    return pl.pallas_call(
        _comb1_body,
        out_shape=[jax.ShapeDtypeStruct((N, HID), jnp.float32),
                   jax.ShapeDtypeStruct((GPAD, HID), jnp.float32)],
    )(h, p, dis, w0, w1, b)


def _comb2_body(h1_ref, q_ref, dis_ref, w0_ref, w1_ref, b_ref, lw_ref, lb_ref, o_ref):
    h1 = h1_ref[...]
    dis = dis_ref[...]
    t2 = q_ref[0, 0:N, :] + q_ref[1, 0:N, :]
    pre = (jax.lax.dot_general(h1, w0_ref[...], (((1,), (0,)), ((), ())),
                               preferred_element_type=jnp.float32)
           - dis * jax.lax.dot_general(t2, w1_ref[...], (((1,), (0,)), ((), ())),
                                       preferred_element_type=jnp.float32)
           + b_ref[...])
    h2 = jnp.maximum(_layer_norm(pre), 0.0)
    o_ref[...] = jax.lax.dot_general(h2, lw_ref[...], (((1,), (0,)), ((), ())),
                                     preferred_element_type=jnp.float32) + lb_ref[...]


def _comb2(h1, q, dis, w0, w1, b, lw, lb):
    return pl.pallas_call(
        _comb2_body,
        out_shape=jax.ShapeDtypeStruct((N, 1), jnp.float32),
    )(h1, q, dis, w0, w1, b, lw, lb)


def _make_scA():
    @functools.partial(
        pl.kernel,
        out_type=[jax.ShapeDtypeStruct((NPAD * EMB,), jnp.float32),
                  jax.ShapeDtypeStruct((NPAD,), jnp.float32)],
        mesh=_mesh(),
        scratch_types=[
            pltpu.VMEM((CE,), jnp.int32),
            pltpu.VMEM((CE,), jnp.int32),
            pltpu.VMEM((LISTCAP,), jnp.int32),
            pltpu.VMEM((LISTCAP,), jnp.int32),
            pltpu.VMEM((LISTCAP,), jnp.int32),
            pltpu.VMEM((KB,), jnp.int32),
            pltpu.VMEM((KB,), jnp.int32),
            pltpu.VMEM((KB,), jnp.int32),
            pltpu.VMEM((KB, EMB), jnp.float32),
            pltpu.VMEM((KB, EMB), jnp.float32),
            pltpu.VMEM((KB, EMB), jnp.float32),
            pltpu.VMEM((NP * EMB,), jnp.float32),
            pltpu.VMEM((NP,), jnp.float32),
            pltpu.SemaphoreType.DMA,
        ],
        compiler_params=_SC_PARAMS,
    )
    def scA(xa_hbm, xb_hbm, eap_hbm, dst_hbm, src_hbm, h_hbm, deg_hbm,
            dchunk, schunk, idlist, dstlist, srclist, ibuf, dbuf, sbuf,
            bufa, bufb, bufe, acc, degacc, sem):
        wid = lax.axis_index("s") * NC + lax.axis_index("c")
        lo = wid * NP
        hi = lo + NP
        iota = lax.iota(jnp.int32, L)
        ones = jnp.ones((L,), jnp.float32)

        def zacc(i, _):
            acc[pl.ds(i * L, L)] = jnp.zeros((L,), jnp.float32)
            return 0
        lax.fori_loop(0, NP * EMB // L, zacc, 0)

        def zdeg(i, _):
            degacc[pl.ds(i * L, L)] = jnp.zeros((L,), jnp.float32)
            return 0
        lax.fori_loop(0, NP // L, zdeg, 0)

        def zlist(i, _):
            z = jnp.zeros((L,), jnp.int32)
            idlist[pl.ds(i * L, L)] = z
            dstlist[pl.ds(i * L, L)] = z
            srclist[pl.ds(i * L, L)] = z
            return 0
        lax.fori_loop(0, LISTCAP // L, zlist, 0)

        def chunk(c, off):
            pltpu.sync_copy(dst_hbm.at[pl.ds(c * CE, CE)], dchunk)
            pltpu.sync_copy(src_hbm.at[pl.ds(c * CE, CE)], schunk)

            def vreg(j, off):
                d = dchunk[pl.ds(j * L, L)]
                s = schunk[pl.ds(j * L, L)]
                mask_s = (s >= lo) & (s < hi) & (s != d)
                cnt_s = plsc.all_reduce_population_count(mask_s)[0]

                @pl.when(cnt_s > 0)
                def _():
                    sidx = jnp.where(mask_s, s - lo, 0)
                    plsc.addupdate_scatter(degacc, [sidx], ones, mask=mask_s)

                mask_d = (d >= lo) & (d < hi)
                cnt_d = plsc.all_reduce_population_count(mask_d)[0]

                @pl.when(cnt_d > 0)
                def _():
                    pos = plsc.cumsum(mask_d.astype(jnp.int32)) - 1 + off
                    pos = jnp.clip(pos, 0, LISTCAP - 1)
                    eid = c * CE + j * L + iota
                    plsc.store_scatter(idlist, [pos], eid, mask=mask_d)
                    plsc.store_scatter(dstlist, [pos], d, mask=mask_d)
                    plsc.store_scatter(srclist, [pos], s, mask=mask_d)

                return jnp.minimum(off + cnt_d, LISTCAP)

            return lax.fori_loop(0, CE // L, vreg, off)

        off = lax.fori_loop(0, E // CE, chunk, jnp.int32(0))

        def blk(b, _):
            def cpy(j, _):
                ibuf[pl.ds(j * L, L)] = idlist[pl.ds(b * KB + j * L, L)]
                dbuf[pl.ds(j * L, L)] = dstlist[pl.ds(b * KB + j * L, L)]
                sbuf[pl.ds(j * L, L)] = srclist[pl.ds(b * KB + j * L, L)]
                return 0
            lax.fori_loop(0, KB // L, cpy, 0)
            c1 = pltpu.async_copy(xa_hbm.at[dbuf], bufa, sem)
            c2 = pltpu.async_copy(xb_hbm.at[sbuf], bufb, sem)
            c3 = pltpu.async_copy(eap_hbm.at[ibuf], bufe, sem)
            c1.wait()
            c2.wait()
            c3.wait()
            cnt = jnp.minimum(off - b * KB, KB)

            def edge(i, _):
                r = plsc.load_gather(dbuf, [jnp.full((L,), i, jnp.int32)])[0] - lo
                z0 = bufa[i, pl.ds(0, L)] + bufb[i, pl.ds(0, L)] + bufe[i, pl.ds(0, L)]
                z1 = bufa[i, pl.ds(L, L)] + bufb[i, pl.ds(L, L)] + bufe[i, pl.ds(L, L)]
                a0 = acc[pl.ds(r * EMB, L)]
                a1 = acc[pl.ds(r * EMB + L, L)]
                acc[pl.ds(r * EMB, L)] = jnp.maximum(a0, z0)
                acc[pl.ds(r * EMB + L, L)] = jnp.maximum(a1, z1)
                return 0

            lax.fori_loop(0, cnt, edge, 0)
            return 0

        nb = (off + KB - 1) // KB
        lax.fori_loop(0, nb, blk, 0)

        pltpu.sync_copy(acc, h_hbm.at[pl.ds(lo * EMB, NP * EMB)])
        pltpu.sync_copy(degacc, deg_hbm.at[pl.ds(lo, NP)])

    return scA


def _make_segsum(D):
    @functools.partial(
        pl.kernel,
        out_type=jax.ShapeDtypeStruct((NC, NPAD, D), jnp.float32),
        mesh=_mesh(),
        scratch_types=[
            pltpu.VMEM((KB,), jnp.int32),
            pltpu.VMEM((KB,), jnp.int32),
            pltpu.VMEM((KB, D), jnp.float32),
            pltpu.VMEM_SHARED((NPAD, D), jnp.float32),
            pltpu.SemaphoreType.DMA,
        ],
        compiler_params=_SC_PARAMS,
    )
    def segsum(g_hbm, src_hbm, dst_hbm, out_hbm, sbuf, dbuf, rows, acc_sh, sem):
        cid = lax.axis_index("c")
        sid = lax.axis_index("s")
        wid = sid * NC + cid

        def zrow(i, _):
            def zc(k, _):
                rows[i, pl.ds(k * L, L)] = jnp.zeros((L,), jnp.float32)
                return 0
            lax.fori_loop(0, D // L, zc, 0)
            return 0
        lax.fori_loop(0, KB, zrow, 0)

        rpt = NPAD // NS

        def zsp(i, _):
            pltpu.sync_copy(rows, acc_sh.at[pl.ds(sid * rpt + i * KB, KB)])
            return 0
        lax.fori_loop(0, rpt // KB, zsp, 0)
        plsc.subcore_barrier()

        nb = (NBLK - wid + NW - 1) // NW

        def blk(k, _):
            b = wid + k * NW
            pltpu.sync_copy(src_hbm.at[pl.ds(b * KB, KB)], sbuf)
            pltpu.sync_copy(dst_hbm.at[pl.ds(b * KB, KB)], dbuf)

            def remap(j, _):
                s = sbuf[pl.ds(j * L, L)]
                d = dbuf[pl.ds(j * L, L)]
                sbuf[pl.ds(j * L, L)] = jnp.where(s == d, jnp.int32(N), s)
                return 0
            lax.fori_loop(0, KB // L, remap, 0)

            pltpu.async_copy(g_hbm.at[sbuf], rows, sem).wait()
            pltpu.sync_copy(rows, acc_sh.at[dbuf], add=True)
            return 0

        lax.fori_loop(0, nb, blk, 0)
        plsc.subcore_barrier()

        def dump(i, _):
            r0 = sid * rpt + i * KB
            pltpu.sync_copy(acc_sh.at[pl.ds(r0, KB)], rows)
            pltpu.sync_copy(rows, out_hbm.at[cid, pl.ds(r0, KB)])
            return 0
        lax.fori_loop(0, rpt // KB, dump, 0)

    return segsum


def kernel(x, edge_index, edge_attr, emb_W, emb_b, c1_W0, c1_W1, c1_b,
           c2_W0, c2_W1, c2_b, lin_W, lin_b):
    src = edge_index[0]
    dst = edge_index[1]

    xa, xb = _node_proj(x, emb_W)
    eap = _edge_proj(edge_attr, emb_W, emb_b)

    hflat, deg = _make_scA()(xa, xb, eap, dst, src)
    h = hflat.reshape(NPAD, EMB)[:N]
    dis, g1p = _mid1(h, deg[:N].reshape(N, 1))

    p = _make_segsum(EMB)(g1p, src, dst)
    h1, g2p = _comb1(h, p, dis, c1_W0, c1_W1, c1_b)

    q = _make_segsum(HID)(g2p, src, dst)
    out = _comb2(h1, q, dis, c2_W0, c2_W1, c2_b, lin_W, lin_b)
    return out[:, 0]
